# baseline (speedup 1.0000x reference)
import functools
import math

import jax
import jax.numpy as jnp
from jax import lax
from jax.experimental import pallas as pl
from jax.experimental.pallas import tpu as pltpu
from jax.experimental.pallas import tpu_sc as plsc

B, P, M, CI, CO = 8, 10000, 16, 3, 64
C = CI * B
GRIDP = -(-P // 512)
NCHUNK = 80
PPAD = NCHUNK * 128
SQ = math.sqrt(0.5)


@functools.cache
def _sc_gather_fn():
    info = plsc.get_sparse_core_info()
    nc, ns = info.num_cores, info.num_subcores
    nw = nc * ns
    tmax = -(-NCHUNK // nw)
    mesh = plsc.VectorSubcoreMesh(core_axis_name="c", subcore_axis_name="s")

    @functools.partial(
        pl.kernel,
        out_type=jax.ShapeDtypeStruct((NCHUNK, M, C, 128), jnp.float32),
        mesh=mesh,
        scratch_types=[
            pltpu.VMEM((M, tmax * 128), jnp.int32),
            pltpu.VMEM((PPAD,), jnp.float32),
            pltpu.VMEM((PPAD,), jnp.float32),
            pltpu.VMEM((M, tmax * 128), jnp.float32),
            pltpu.SemaphoreType.DMA,
            pltpu.SemaphoreType.DMA,
        ],
        compiler_params=pltpu.CompilerParams(needs_layout_passes=False),
    )
    def sc_gather(tab_hbm, idx_hbm, out_hbm, idx_v, col0_v, col1_v, h_v,
                  sem0, sem1):
        wid = lax.axis_index("s") * nc + lax.axis_index("c")

        for t in range(tmax):
            chunk = wid + t * nw

            @pl.when(chunk < NCHUNK)
            def _(t=t, chunk=chunk):
                pltpu.sync_copy(idx_hbm.at[:, pl.ds(chunk * 128, 128)],
                                idx_v.at[:, pl.ds(t * 128, 128)])

        def gather_col(c, col_v):
            def do_m(m, carry2):
                for t in range(tmax):
                    chunk = wid + t * nw

                    @pl.when(chunk < NCHUNK)
                    def _(t=t, m=m):
                        for l in range(8):
                            off = t * 128 + l * 16
                            i16 = idx_v[m, pl.ds(off, 16)]
                            h_v[m, pl.ds(off, 16)] = plsc.load_gather(
                                col_v, [i16])
                return carry2

            lax.fori_loop(0, M, do_m, jnp.int32(0))
            for t in range(tmax):
                chunk = wid + t * nw

                @pl.when(chunk < NCHUNK)
                def _(t=t, chunk=chunk, c=c):
                    pltpu.sync_copy(h_v.at[:, pl.ds(t * 128, 128)],
                                    out_hbm.at[chunk, :, c, :])

        pltpu.async_copy(tab_hbm.at[jnp.int32(0)], col0_v, sem0)

        def do_pair(c2, carry):
            pltpu.async_copy(tab_hbm.at[jnp.int32(c2 + 1)], col1_v, sem1)
            pltpu.make_async_copy(tab_hbm.at[jnp.int32(c2)], col0_v, sem0).wait()
            gather_col(jnp.int32(c2), col0_v)

            @pl.when(c2 + 2 < C)
            def _():
                pltpu.async_copy(tab_hbm.at[jnp.int32(c2 + 2)], col0_v, sem0)

            pltpu.make_async_copy(tab_hbm.at[jnp.int32(c2 + 1)], col1_v, sem1).wait()
            gather_col(jnp.int32(c2 + 1), col1_v)
            return carry

        for c2 in range(0, C, 2):
            do_pair(c2, jnp.int32(0))

    return sc_gather


def _tc_body(w_ref, h_ref, x_ref, wr_ref, b_ref, o_ref):
    bias = jnp.broadcast_to(b_ref[...], (CO, 128))
    wr_cols = [jnp.broadcast_to(wr_ref[:, i][:, None], (CO, 128)) for i in range(CI)]
    zero = jnp.zeros((CO, 128), jnp.float32)
    for j in range(4):
        sl = pl.ds(j * 128, 128)
        for half in range(4):
            bs = [2 * half + t for t in range(2)]
            acc = [zero] * 2
            for m in range(M):
                for i in range(CI):
                    wv = w_ref[m, i, :, sl]
                    for t, b in enumerate(bs):
                        acc[t] = acc[t] + wv * h_ref[j, m, i * 8 + b, :][None, :]
            for t, b in enumerate(bs):
                av = acc[t] + bias
                conv = jnp.where(av > 0, av, jnp.exp(av) - 1.0)
                res = zero
                for i in range(CI):
                    res = res + wr_cols[i] * x_ref[i, b, sl][None, :]
                o_ref[b, :, sl] = SQ * conv + SQ * res


def _tc_compute(wt, h, xt, wr, bias2):
    return pl.pallas_call(
        _tc_body,
        grid=(GRIDP,),
        in_specs=[
            pl.BlockSpec((M, CI, CO, 512), lambda k: (k * 0, k * 0, k * 0, k)),
            pl.BlockSpec((4, M, C, 128), lambda k: (k, k * 0, k * 0, k * 0)),
            pl.BlockSpec((CI, B, 512), lambda k: (k * 0, k * 0, k)),
            pl.BlockSpec((CO, CI), lambda k: (k * 0, k * 0)),
            pl.BlockSpec((CO, 1), lambda k: (k * 0, k * 0)),
        ],
        out_specs=pl.BlockSpec((B, CO, 512), lambda k: (k * 0, k * 0, k)),
        out_shape=jax.ShapeDtypeStruct((B, CO, P), jnp.float32),
        compiler_params=pltpu.CompilerParams(
            dimension_semantics=("parallel",),
        ),
    )(wt, h, xt, wr, bias2)


def kernel(in_pc, neighbor_id, weights, bias, weight_res):
    in_pc = in_pc.astype(jnp.float32)
    weights = weights.astype(jnp.float32)
    tab = jnp.transpose(in_pc, (2, 0, 1)).reshape(C, P)
    tab = jnp.pad(tab, ((0, 0), (0, PPAD - P)))
    idx = jnp.transpose(neighbor_id.astype(jnp.int32), (1, 0))
    idx = jnp.pad(idx, ((0, 0), (0, PPAD - P)))
    h = _sc_gather_fn()(tab, idx)
    wt = jnp.transpose(weights, (1, 3, 2, 0))
    xt = jnp.transpose(in_pc, (2, 0, 1))
    out_t = _tc_compute(wt, h, xt, weight_res.astype(jnp.float32),
                        bias.astype(jnp.float32)[:, None])
    return jnp.transpose(out_t, (0, 2, 1)).astype(jnp.float64)


# speedup vs baseline: 1.0112x; 1.0112x over previous
import functools
import math

import jax
import jax.numpy as jnp
from jax import lax
from jax.experimental import pallas as pl
from jax.experimental.pallas import tpu as pltpu
from jax.experimental.pallas import tpu_sc as plsc

B, P, M, CI, CO = 8, 10000, 16, 3, 64
C = CI * B
GRIDP = -(-P // 512)
NCHUNK = 80
PPAD = NCHUNK * 128
SQ = math.sqrt(0.5)


@functools.cache
def _sc_gather_fn():
    info = plsc.get_sparse_core_info()
    nc, ns = info.num_cores, info.num_subcores
    nw = nc * ns
    tmax = -(-NCHUNK // nw)
    mesh = plsc.VectorSubcoreMesh(core_axis_name="c", subcore_axis_name="s")

    @functools.partial(
        pl.kernel,
        out_type=jax.ShapeDtypeStruct((NCHUNK, M, C, 128), jnp.float32),
        mesh=mesh,
        scratch_types=[
            pltpu.VMEM((M, tmax * 128), jnp.int32),
            pltpu.VMEM((PPAD,), jnp.float32),
            pltpu.VMEM((PPAD,), jnp.float32),
            pltpu.VMEM((M, tmax * 128), jnp.float32),
            pltpu.VMEM((M, tmax * 128), jnp.float32),
            pltpu.SemaphoreType.DMA,
            pltpu.SemaphoreType.DMA,
            pltpu.SemaphoreType.DMA,
            pltpu.SemaphoreType.DMA,
        ],
        compiler_params=pltpu.CompilerParams(needs_layout_passes=False),
    )
    def sc_gather(tab_hbm, idx_hbm, out_hbm, idx_v, col0_v, col1_v,
                  h0_v, h1_v, sem0, sem1, sem2, sem3):
        wid = lax.axis_index("s") * nc + lax.axis_index("c")

        for t in range(tmax):
            chunk = wid + t * nw

            @pl.when(chunk < NCHUNK)
            def _(t=t, chunk=chunk):
                pltpu.sync_copy(idx_hbm.at[:, pl.ds(chunk * 128, 128)],
                                idx_v.at[:, pl.ds(t * 128, 128)])

        def gather_col(c, cprev, col_v, h_v, semh):
            if cprev is not None:
                for t in range(tmax):
                    chunk = wid + t * nw

                    @pl.when(chunk < NCHUNK)
                    def _(t=t, chunk=chunk):
                        pltpu.make_async_copy(
                            h_v.at[:, pl.ds(t * 128, 128)],
                            out_hbm.at[chunk, :, jnp.int32(cprev), :],
                            semh).wait()

            def do_m(m, carry2):
                for t in range(tmax):
                    chunk = wid + t * nw

                    @pl.when(chunk < NCHUNK)
                    def _(t=t, m=m):
                        for l in range(8):
                            off = t * 128 + l * 16
                            i16 = idx_v[m, pl.ds(off, 16)]
                            h_v[m, pl.ds(off, 16)] = plsc.load_gather(
                                col_v, [i16])
                return carry2

            lax.fori_loop(0, M, do_m, jnp.int32(0))
            for t in range(tmax):
                chunk = wid + t * nw

                @pl.when(chunk < NCHUNK)
                def _(t=t, chunk=chunk, c=c):
                    pltpu.async_copy(h_v.at[:, pl.ds(t * 128, 128)],
                                     out_hbm.at[chunk, :, jnp.int32(c), :],
                                     semh)

        pltpu.async_copy(tab_hbm.at[jnp.int32(0)], col0_v, sem0)

        def do_pair(c2, carry):
            pltpu.async_copy(tab_hbm.at[jnp.int32(c2 + 1)], col1_v, sem1)
            pltpu.make_async_copy(tab_hbm.at[jnp.int32(c2)], col0_v, sem0).wait()
            gather_col(c2, c2 - 2 if c2 >= 2 else None, col0_v, h0_v, sem2)

            @pl.when(c2 + 2 < C)
            def _():
                pltpu.async_copy(tab_hbm.at[jnp.int32(c2 + 2)], col0_v, sem0)

            pltpu.make_async_copy(tab_hbm.at[jnp.int32(c2 + 1)], col1_v, sem1).wait()
            gather_col(c2 + 1, c2 - 1 if c2 >= 1 else None, col1_v, h1_v, sem3)
            return carry

        for c2 in range(0, C, 2):
            do_pair(c2, jnp.int32(0))
        for h_v, semh, clast in ((h0_v, sem2, C - 2), (h1_v, sem3, C - 1)):
            for t in range(tmax):
                chunk = wid + t * nw

                @pl.when(chunk < NCHUNK)
                def _(t=t, chunk=chunk, h_v=h_v, semh=semh, clast=clast):
                    pltpu.make_async_copy(
                        h_v.at[:, pl.ds(t * 128, 128)],
                        out_hbm.at[chunk, :, jnp.int32(clast), :],
                        semh).wait()

    return sc_gather


def _tc_body(w_ref, h_ref, x_ref, wr_ref, b_ref, o_ref):
    bias = jnp.broadcast_to(b_ref[...], (CO, 128))
    wr_cols = [jnp.broadcast_to(wr_ref[:, i][:, None], (CO, 128)) for i in range(CI)]
    zero = jnp.zeros((CO, 128), jnp.float32)
    for j in range(4):
        sl = pl.ds(j * 128, 128)
        for half in range(4):
            bs = [2 * half + t for t in range(2)]
            acc = [zero] * 2
            for m in range(M):
                for i in range(CI):
                    wv = w_ref[m, i, :, sl]
                    for t, b in enumerate(bs):
                        acc[t] = acc[t] + wv * h_ref[j, m, i * 8 + b, :][None, :]
            for t, b in enumerate(bs):
                av = acc[t] + bias
                conv = jnp.where(av > 0, av, jnp.exp(av) - 1.0)
                res = zero
                for i in range(CI):
                    res = res + wr_cols[i] * x_ref[i, b, sl][None, :]
                o_ref[b, :, sl] = SQ * conv + SQ * res


def _tc_compute(wt, h, xt, wr, bias2):
    return pl.pallas_call(
        _tc_body,
        grid=(GRIDP,),
        in_specs=[
            pl.BlockSpec((M, CI, CO, 512), lambda k: (k * 0, k * 0, k * 0, k)),
            pl.BlockSpec((4, M, C, 128), lambda k: (k, k * 0, k * 0, k * 0)),
            pl.BlockSpec((CI, B, 512), lambda k: (k * 0, k * 0, k)),
            pl.BlockSpec((CO, CI), lambda k: (k * 0, k * 0)),
            pl.BlockSpec((CO, 1), lambda k: (k * 0, k * 0)),
        ],
        out_specs=pl.BlockSpec((B, CO, 512), lambda k: (k * 0, k * 0, k)),
        out_shape=jax.ShapeDtypeStruct((B, CO, P), jnp.float32),
        compiler_params=pltpu.CompilerParams(
            dimension_semantics=("parallel",),
        ),
    )(wt, h, xt, wr, bias2)


def kernel(in_pc, neighbor_id, weights, bias, weight_res):
    in_pc = in_pc.astype(jnp.float32)
    weights = weights.astype(jnp.float32)
    tab = jnp.transpose(in_pc, (2, 0, 1)).reshape(C, P)
    tab = jnp.pad(tab, ((0, 0), (0, PPAD - P)))
    idx = jnp.transpose(neighbor_id.astype(jnp.int32), (1, 0))
    idx = jnp.pad(idx, ((0, 0), (0, PPAD - P)))
    h = _sc_gather_fn()(tab, idx)
    wt = jnp.transpose(weights, (1, 3, 2, 0))
    xt = jnp.transpose(in_pc, (2, 0, 1))
    out_t = _tc_compute(wt, h, xt, weight_res.astype(jnp.float32),
                        bias.astype(jnp.float32)[:, None])
    return jnp.transpose(out_t, (0, 2, 1)).astype(jnp.float64)
